# Initial kernel scaffold; baseline (speedup 1.0000x reference)
import jax
import jax.numpy as jnp
from jax.experimental import pallas as pl


def kernel(hidden_states, cand_emb_index, W_dense, b_dense, ln_gamma, ln_beta, decoder_table, entity_bias):
    raise NotImplementedError("write your pallas kernel here")


# speedup vs baseline: 17.4312x; 17.4312x over previous
import functools

import jax
import jax.numpy as jnp
from jax import lax
from jax.experimental import pallas as pl
from jax.experimental.pallas import tpu as pltpu
from jax.experimental.pallas import tpu_sc as plsc

LN_EPS = 1e-12

_NC = 2
_NS = 16
_L = 16
_NW = _NC * _NS


def _transform_body(x_ref, w_ref, b_ref, g_ref, beta_ref, o_ref):
    h = jnp.dot(x_ref[...], w_ref[...], preferred_element_type=jnp.float32)
    h = h + b_ref[...]
    h = 0.5 * h * (1.0 + lax.erf(h * (2.0 ** -0.5)))
    mu = jnp.mean(h, axis=-1, keepdims=True)
    var = jnp.mean((h - mu) ** 2, axis=-1, keepdims=True)
    h = (h - mu) * lax.rsqrt(var + LN_EPS) * g_ref[...] + beta_ref[...]
    o_ref[...] = h


def _transform(hidden_states, W_dense, b_dense, ln_gamma, ln_beta):
    B, H = hidden_states.shape
    D = W_dense.shape[1]
    BM = 512
    grid = (B // BM,)
    return pl.pallas_call(
        _transform_body,
        grid=grid,
        in_specs=[
            pl.BlockSpec((BM, H), lambda i: (i, 0)),
            pl.BlockSpec((H, D), lambda i: (0, 0)),
            pl.BlockSpec((1, D), lambda i: (0, 0)),
            pl.BlockSpec((1, D), lambda i: (0, 0)),
            pl.BlockSpec((1, D), lambda i: (0, 0)),
        ],
        out_specs=pl.BlockSpec((BM, D), lambda i: (i, 0)),
        out_shape=jax.ShapeDtypeStruct((B, D), jnp.float32),
    )(
        hidden_states,
        W_dense,
        b_dense.reshape(1, D),
        ln_gamma.reshape(1, D),
        ln_beta.reshape(1, D),
    )


_GATHER_DNUMS = lax.GatherDimensionNumbers(
    offset_dims=(), collapsed_slice_dims=(0,), start_index_map=(0,))


def _shuffle(v, idx):
    return lax.gather(v, idx[:, None], _GATHER_DNUMS, (1,),
                      mode=lax.GatherScatterMode.PROMISE_IN_BOUNDS)


def _lane_total(v, rot_idx):
    for idx in rot_idx:
        v = v + _shuffle(v, idx)
    return v


def _make_score_kernel(B, C, D, V):
    b_per_w = B // _NW
    n_chunks = D // _L
    n_groups = C // _L
    mesh = plsc.VectorSubcoreMesh(core_axis_name="c", subcore_axis_name="s")

    @functools.partial(
        pl.kernel,
        mesh=mesh,
        out_type=jax.ShapeDtypeStruct((B, C), jnp.float32),
        scratch_types=[
            pltpu.VMEM((C,), jnp.int32),
            pltpu.VMEM((C, D), jnp.float32),
            pltpu.VMEM((C,), jnp.float32),
            pltpu.VMEM((D,), jnp.float32),
            pltpu.VMEM((C,), jnp.float32),
            pltpu.SemaphoreType.DMA,
        ],
    )
    def score_kernel(h_hbm, idx_hbm, table_hbm, bias_hbm, out_hbm,
                     idx_v, rows_v, bias_v, h_v, scores_v, sem):
        wid = lax.axis_index("s") * _NC + lax.axis_index("c")
        base = wid * b_per_w
        lane = lax.iota(jnp.int32, _L)
        rot_idx = [(lane + sh) % _L for sh in (8, 4, 2, 1)]
        lane_masks = [lane == j for j in range(_L)]

        def body(i, carry):
            b = base + i
            pltpu.sync_copy(idx_hbm.at[b], idx_v)
            pltpu.sync_copy(h_hbm.at[b], h_v)
            pltpu.async_copy(table_hbm.at[idx_v], rows_v, sem).wait()
            pltpu.async_copy(bias_hbm.at[idx_v], bias_v, sem).wait()
            hs = [h_v[pl.ds(k * _L, _L)] for k in range(n_chunks)]

            def cgroup(g, carry2):
                c0 = g * _L
                svec = jnp.zeros((_L,), jnp.float32)
                for j in range(_L):
                    acc = rows_v[c0 + j, pl.ds(0, _L)] * hs[0]
                    for k in range(1, n_chunks):
                        acc = acc + rows_v[c0 + j, pl.ds(k * _L, _L)] * hs[k]
                    tot = _lane_total(acc, rot_idx)
                    svec = jnp.where(lane_masks[j], tot, svec)
                scores_v[pl.ds(c0, _L)] = svec + bias_v[pl.ds(c0, _L)]
                return carry2

            lax.fori_loop(0, n_groups, cgroup, 0)
            pltpu.sync_copy(scores_v, out_hbm.at[b])
            return carry

        lax.fori_loop(0, b_per_w, body, 0)

    return score_kernel


def kernel(hidden_states, cand_emb_index, W_dense, b_dense, ln_gamma,
           ln_beta, decoder_table, entity_bias):
    B, H = hidden_states.shape
    V, D = decoder_table.shape
    C = cand_emb_index.shape[1]
    idx = cand_emb_index.astype(jnp.int32)
    h = _transform(hidden_states, W_dense, b_dense, ln_gamma, ln_beta)
    score = _make_score_kernel(B, C, D, V)
    return score(h, idx, decoder_table, entity_bias)


# speedup vs baseline: 35.2929x; 2.0247x over previous
import functools

import jax
import jax.numpy as jnp
from jax import lax
from jax.experimental import pallas as pl
from jax.experimental.pallas import tpu as pltpu
from jax.experimental.pallas import tpu_sc as plsc

LN_EPS = 1e-12

_NC = 2
_NS = 16
_L = 16
_NW = _NC * _NS


def _transform_body(x_ref, w_ref, b_ref, g_ref, beta_ref, o_ref):
    h = jnp.dot(x_ref[...], w_ref[...], preferred_element_type=jnp.float32)
    h = h + b_ref[...]
    h = 0.5 * h * (1.0 + lax.erf(h * (2.0 ** -0.5)))
    mu = jnp.mean(h, axis=-1, keepdims=True)
    var = jnp.mean((h - mu) ** 2, axis=-1, keepdims=True)
    h = (h - mu) * lax.rsqrt(var + LN_EPS) * g_ref[...] + beta_ref[...]
    o_ref[...] = h


def _transform(hidden_states, W_dense, b_dense, ln_gamma, ln_beta):
    B, H = hidden_states.shape
    D = W_dense.shape[1]
    BM = 512
    grid = (B // BM,)
    return pl.pallas_call(
        _transform_body,
        grid=grid,
        in_specs=[
            pl.BlockSpec((BM, H), lambda i: (i, 0)),
            pl.BlockSpec((H, D), lambda i: (0, 0)),
            pl.BlockSpec((1, D), lambda i: (0, 0)),
            pl.BlockSpec((1, D), lambda i: (0, 0)),
            pl.BlockSpec((1, D), lambda i: (0, 0)),
        ],
        out_specs=pl.BlockSpec((BM, D), lambda i: (i, 0)),
        out_shape=jax.ShapeDtypeStruct((B, D), jnp.float32),
    )(
        hidden_states,
        W_dense,
        b_dense.reshape(1, D),
        ln_gamma.reshape(1, D),
        ln_beta.reshape(1, D),
    )


_GATHER_DNUMS = lax.GatherDimensionNumbers(
    offset_dims=(), collapsed_slice_dims=(0,), start_index_map=(0,))


def _shuffle(v, idx):
    return lax.gather(v, idx[:, None], _GATHER_DNUMS, (1,),
                      mode=lax.GatherScatterMode.PROMISE_IN_BOUNDS)


def _lane_total(v, rot_idx):
    for idx in rot_idx:
        v = v + _shuffle(v, idx)
    return v


def _make_score_kernel(B, C, D, V):
    b_per_w = B // _NW
    n_chunks = D // _L
    n_groups = C // _L
    CH = 16
    n_pairs = b_per_w // 2
    pairs_per_ch = CH // 2
    n_ch = b_per_w // CH
    mesh = plsc.VectorSubcoreMesh(core_axis_name="c", subcore_axis_name="s")

    @functools.partial(
        pl.kernel,
        mesh=mesh,
        out_type=jax.ShapeDtypeStruct((B, C), jnp.float32),
        scratch_types=[
            pltpu.VMEM((2, CH, C), jnp.int32),
            pltpu.VMEM((2, CH, D), jnp.float32),
            pltpu.VMEM((2, C, D), jnp.float32),
            pltpu.VMEM((2, C), jnp.float32),
            pltpu.VMEM((CH, C), jnp.float32),
            pltpu.SemaphoreType.DMA,
            pltpu.SemaphoreType.DMA,
            pltpu.SemaphoreType.DMA,
        ],
    )
    def score_kernel(h_hbm, idx_hbm, table_hbm, bias_hbm, out_hbm,
                     idx_c, h_c, rows_v, bias_v, scores_v,
                     gsem0, gsem1, psem):
        wid = lax.axis_index("s") * _NC + lax.axis_index("c")
        base = wid * b_per_w
        lane = lax.iota(jnp.int32, _L)
        rot_idx = [(lane + sh) % _L for sh in (8, 4, 2, 1)]
        lane_masks = [lane == j for j in range(_L)]
        gsems = (gsem0, gsem1)

        def issue_gather(off, p):
            idx_ref = idx_c.at[(off >> 4) & 1, off & (CH - 1)]
            pltpu.async_copy(table_hbm.at[idx_ref], rows_v.at[p], gsems[p])
            pltpu.async_copy(bias_hbm.at[idx_ref], bias_v.at[p], gsems[p])

        def wait_gather(p):
            idx_ref = idx_c.at[0, 0]
            pltpu.make_async_copy(
                table_hbm.at[idx_ref], rows_v.at[p], gsems[p]).wait()
            pltpu.make_async_copy(
                bias_hbm.at[idx_ref], bias_v.at[p], gsems[p]).wait()

        pltpu.sync_copy(idx_hbm.at[pl.ds(base, CH)], idx_c.at[0])
        pltpu.sync_copy(h_hbm.at[pl.ds(base, CH)], h_c.at[0])
        issue_gather(0, 0)
        issue_gather(1, 1)

        def pair_body(t, carry):
            s = t >> 3

            @pl.when((t & (pairs_per_ch - 1)) == 0)
            def _prefetch_next_chunk():
                s_next = jnp.minimum(s + 1, n_ch - 1)
                q = (s + 1) & 1
                pltpu.async_copy(
                    idx_hbm.at[pl.ds(base + s_next * CH, CH)],
                    idx_c.at[q], psem)
                pltpu.async_copy(
                    h_hbm.at[pl.ds(base + s_next * CH, CH)],
                    h_c.at[q], psem)

            @pl.when((t & (pairs_per_ch - 1)) == pairs_per_ch - 1)
            def _wait_next_chunk():
                pltpu.make_async_copy(
                    idx_hbm.at[pl.ds(base, CH)], idx_c.at[0], psem).wait()
                pltpu.make_async_copy(
                    h_hbm.at[pl.ds(base, CH)], h_c.at[0], psem).wait()

            sbuf = s & 1
            for p in (0, 1):
                off = 2 * t + p
                r = off & (CH - 1)
                wait_gather(p)
                hs = [h_c[sbuf, r, pl.ds(k * _L, _L)]
                      for k in range(n_chunks)]

                def cgroup(g, carry2, _p=p, _r=r, _hs=hs):
                    c0 = g * _L
                    svec = jnp.zeros((_L,), jnp.float32)
                    for j in range(_L):
                        acc = rows_v[_p, c0 + j, pl.ds(0, _L)] * _hs[0]
                        for k in range(1, n_chunks):
                            acc = acc + (rows_v[_p, c0 + j,
                                                pl.ds(k * _L, _L)] * _hs[k])
                        tot = _lane_total(acc, rot_idx)
                        svec = jnp.where(lane_masks[j], tot, svec)
                    scores_v[_r, pl.ds(c0, _L)] = (
                        svec + bias_v[_p, pl.ds(c0, _L)])
                    return carry2

                lax.fori_loop(0, n_groups, cgroup, 0)
                issue_gather(jnp.minimum(off + 2, b_per_w - 1), p)

            @pl.when((t & (pairs_per_ch - 1)) == pairs_per_ch - 1)
            def _store_scores():
                pltpu.sync_copy(scores_v,
                                out_hbm.at[pl.ds(base + s * CH, CH)])

            return carry

        lax.fori_loop(0, n_pairs, pair_body, 0)
        wait_gather(0)
        wait_gather(1)

    return score_kernel


def kernel(hidden_states, cand_emb_index, W_dense, b_dense, ln_gamma,
           ln_beta, decoder_table, entity_bias):
    B, H = hidden_states.shape
    V, D = decoder_table.shape
    C = cand_emb_index.shape[1]
    idx = cand_emb_index.astype(jnp.int32)
    h = _transform(hidden_states, W_dense, b_dense, ln_gamma, ln_beta)
    score = _make_score_kernel(B, C, D, V)
    return score(h, idx, decoder_table, entity_bias)
